# Initial kernel scaffold; baseline (speedup 1.0000x reference)
import jax
import jax.numpy as jnp
from jax.experimental import pallas as pl


def kernel(x, expert_indices, w1, w2, w3):
    raise NotImplementedError("write your pallas kernel here")


# speedup vs baseline: 3.4732x; 3.4732x over previous
import jax
import jax.numpy as jnp
from jax.experimental import pallas as pl
from jax.experimental.pallas import tpu as pltpu

_T, _A, _E, _D, _F = 8, 2, 8, 1024, 2816
_FT = 256
_NF = _F // _FT


def _ffn_kernel(idx_ref, x_ref, w1_ref, w2_ref, w3_ref, out_ref, acc_ref):
    e = pl.program_id(0)
    f = pl.program_id(1)

    @pl.when(f == 0)
    def _init():
        acc_ref[...] = jnp.zeros_like(acc_ref)

    xb = x_ref[...]
    w1b = w1_ref[0]
    w3b = w3_ref[0]
    w2b = w2_ref[0]
    dims = (((1,), (1,)), ((), ()))
    x1 = jax.lax.dot_general(xb, w1b, dims,
                             preferred_element_type=jnp.float32)
    x3 = jax.lax.dot_general(xb, w3b, dims,
                             preferred_element_type=jnp.float32)
    h = (x1 * jax.nn.sigmoid(x1)) * x3
    acc_ref[...] += jax.lax.dot_general(h, w2b, dims,
                                        preferred_element_type=jnp.float32)

    @pl.when(f == _NF - 1)
    def _scatter():
        for p in range(_T * _A):
            t = p // _A

            @pl.when(idx_ref[p] == e)
            def _write():
                out_ref[p, :] = acc_ref[t, :]


def kernel(x, expert_indices, w1, w2, w3):
    idx = expert_indices.reshape(-1).astype(jnp.int32)
    grid_spec = pltpu.PrefetchScalarGridSpec(
        num_scalar_prefetch=1,
        grid=(_E, _NF),
        in_specs=[
            pl.BlockSpec((_T, _D), lambda e, f, idx_ref: (0, 0)),
            pl.BlockSpec((1, _FT, _D), lambda e, f, idx_ref: (e, f, 0)),
            pl.BlockSpec((1, _D, _FT), lambda e, f, idx_ref: (e, 0, f)),
            pl.BlockSpec((1, _FT, _D), lambda e, f, idx_ref: (e, f, 0)),
        ],
        out_specs=pl.BlockSpec((_T * _A, _D), lambda e, f, idx_ref: (0, 0)),
        scratch_shapes=[pltpu.VMEM((_T, _D), jnp.float32)],
    )
    out = pl.pallas_call(
        _ffn_kernel,
        grid_spec=grid_spec,
        out_shape=jax.ShapeDtypeStruct((_T * _A, _D), jnp.float32),
    )(idx, x, w1, w2, w3)
    return out.reshape(_T, _A, _D)


# speedup vs baseline: 4.7108x; 1.3563x over previous
import jax
import jax.numpy as jnp
from jax.experimental import pallas as pl
from jax.experimental.pallas import tpu as pltpu

_T, _A, _E, _D, _F = 8, 2, 8, 1024, 2816
_FT = 1408
_NF = _F // _FT


def _ffn_kernel(idx_ref, x_ref, w1_ref, w2_ref, w3_ref, out_ref, acc_ref):
    e = pl.program_id(0)
    f = pl.program_id(1)

    @pl.when(f == 0)
    def _init():
        acc_ref[...] = jnp.zeros_like(acc_ref)

    xb = x_ref[...]
    w1b = w1_ref[0]
    w3b = w3_ref[0]
    w2b = w2_ref[0]
    dims = (((1,), (1,)), ((), ()))
    x1 = jax.lax.dot_general(xb, w1b, dims,
                             preferred_element_type=jnp.float32)
    x3 = jax.lax.dot_general(xb, w3b, dims,
                             preferred_element_type=jnp.float32)
    h = (x1 * jax.nn.sigmoid(x1)) * x3
    acc_ref[...] += jax.lax.dot_general(h, w2b, dims,
                                        preferred_element_type=jnp.float32)

    @pl.when(f == _NF - 1)
    def _scatter():
        for p in range(_T * _A):
            t = p // _A

            @pl.when(idx_ref[p] == e)
            def _write():
                out_ref[p, :] = acc_ref[t, :]


def kernel(x, expert_indices, w1, w2, w3):
    idx = expert_indices.reshape(-1).astype(jnp.int32)
    grid_spec = pltpu.PrefetchScalarGridSpec(
        num_scalar_prefetch=1,
        grid=(_E, _NF),
        in_specs=[
            pl.BlockSpec((_T, _D), lambda e, f, idx_ref: (0, 0)),
            pl.BlockSpec((1, _FT, _D), lambda e, f, idx_ref: (e, f, 0)),
            pl.BlockSpec((1, _D, _FT), lambda e, f, idx_ref: (e, 0, f)),
            pl.BlockSpec((1, _FT, _D), lambda e, f, idx_ref: (e, f, 0)),
        ],
        out_specs=pl.BlockSpec((_T * _A, _D), lambda e, f, idx_ref: (0, 0)),
        scratch_shapes=[pltpu.VMEM((_T, _D), jnp.float32)],
    )
    out = pl.pallas_call(
        _ffn_kernel,
        grid_spec=grid_spec,
        out_shape=jax.ShapeDtypeStruct((_T * _A, _D), jnp.float32),
    )(idx, x, w1, w2, w3)
    return out.reshape(_T, _A, _D)


# speedup vs baseline: 6.7047x; 1.4233x over previous
import jax
import jax.numpy as jnp
from jax.experimental import pallas as pl
from jax.experimental.pallas import tpu as pltpu

_T, _A, _E, _D, _F = 8, 2, 8, 1024, 2816
_FT = 1408
_NF = _F // _FT


def _ffn_kernel(idx_ref, meta_ref, x_ref, w1_ref, w2_ref, w3_ref, out_ref,
                acc_ref):
    e = pl.program_id(0)
    f = pl.program_id(1)
    expert = meta_ref[e]
    valid = e < meta_ref[_E]

    @pl.when(valid)
    def _body():
        @pl.when(f == 0)
        def _init():
            acc_ref[...] = jnp.zeros_like(acc_ref)

        xb = x_ref[...]
        w1b = w1_ref[0]
        w3b = w3_ref[0]
        w2b = w2_ref[0]
        dims = (((1,), (1,)), ((), ()))
        x1 = jax.lax.dot_general(xb, w1b, dims,
                                 preferred_element_type=jnp.float32)
        x3 = jax.lax.dot_general(xb, w3b, dims,
                                 preferred_element_type=jnp.float32)
        h = (x1 * jax.nn.sigmoid(x1)) * x3
        acc_ref[...] += jax.lax.dot_general(h, w2b, dims,
                                            preferred_element_type=jnp.float32)

        @pl.when(f == _NF - 1)
        def _scatter():
            for p in range(_T * _A):
                t = p // _A

                @pl.when(idx_ref[p] == expert)
                def _write():
                    out_ref[p, :] = acc_ref[t, :]


def kernel(x, expert_indices, w1, w2, w3):
    idx = expert_indices.reshape(-1).astype(jnp.int32)
    present = jnp.zeros((_E,), jnp.bool_).at[idx].set(True)
    n_used = jnp.sum(present.astype(jnp.int32))
    order = jnp.argsort(jnp.logical_not(present)).astype(jnp.int32)
    sched = order[jnp.minimum(jnp.arange(_E), n_used - 1)]
    meta = jnp.concatenate([sched, n_used[None]])

    def _w_fd(e, f, idx_ref, m):
        return (m[e], jnp.where(e < m[_E], f, _NF - 1), 0)

    def _w_df(e, f, idx_ref, m):
        return (m[e], 0, jnp.where(e < m[_E], f, _NF - 1))

    grid_spec = pltpu.PrefetchScalarGridSpec(
        num_scalar_prefetch=2,
        grid=(_E, _NF),
        in_specs=[
            pl.BlockSpec((_T, _D), lambda e, f, i, m: (0, 0)),
            pl.BlockSpec((1, _FT, _D), _w_fd),
            pl.BlockSpec((1, _D, _FT), _w_df),
            pl.BlockSpec((1, _FT, _D), _w_fd),
        ],
        out_specs=pl.BlockSpec((_T * _A, _D), lambda e, f, i, m: (0, 0)),
        scratch_shapes=[pltpu.VMEM((_T, _D), jnp.float32)],
    )
    out = pl.pallas_call(
        _ffn_kernel,
        grid_spec=grid_spec,
        out_shape=jax.ShapeDtypeStruct((_T * _A, _D), jnp.float32),
    )(idx, meta, x, w1, w2, w3)
    return out.reshape(_T, _A, _D)
